# baseline (speedup 1.0000x reference)
import numpy as np
import jax
import jax.numpy as jnp
from jax.experimental import pallas as pl
from jax.experimental.pallas import tpu as pltpu

_K0 = 928981903
_K1 = 3453687069
_K2 = (_K0 ^ _K1 ^ 0x1BD11BDA) & 0xFFFFFFFF

_B = 32
_V = 1_000_000
_CHUNK = 65536
_SUB = 512
_NSUB = _CHUNK // _SUB
_UNROLL = 32
_GRID = (_V + _CHUNK - 1) // _CHUNK

_ROT_A = (13, 15, 26, 6)
_ROT_B = (17, 29, 16, 24)

_KS = (_K0, _K1, _K2)
_SCHED = tuple(
    (np.uint32(_KS[(g + 1) % 3]), np.uint32((_KS[(g + 2) % 3] + g + 1) & 0xFFFFFFFF))
    for g in range(5)
)

_A_NP = (
    np.arange(_B, dtype=np.uint64)[:, None] * _V
    + np.arange(_SUB, dtype=np.uint64)[None, :]
    + _K1
) & 0xFFFFFFFF
_A_CONST = _A_NP.astype(np.uint32)
_COL_CONST = np.broadcast_to(
    np.arange(_SUB, dtype=np.int32)[None, :], (_B, _SUB)
).copy()


def _gumbel_from_counter(x1):
    x0 = x1 + jnp.uint32(_K0)
    first = True
    for g in range(5):
        rots = _ROT_A if g % 2 == 0 else _ROT_B
        for r in rots:
            if first:
                first = False
            else:
                x0 = x0 + x1
            t = (x1 << jnp.uint32(r)) | (x1 >> jnp.uint32(32 - r))
            x1 = x0 ^ t
        c0, c1 = _SCHED[g]
        x0 = x0 + c0
        x1 = x1 + c1
    bits = x0 ^ x1
    fb = (bits >> jnp.uint32(9)) | jnp.uint32(0x3F800000)
    f = jax.lax.bitcast_convert_type(fb, jnp.float32) - jnp.float32(1.0)
    u = f + jnp.float32(1e-20)
    return -jnp.log(-jnp.log(u))


def _sample_kernel(x_ref, a_ref, col_ref, out_ref, accv, acct):
    i = pl.program_id(0)

    @pl.when(i == 0)
    def _():
        accv[...] = jnp.full((_B, _SUB), -jnp.inf, jnp.float32)
        acct[...] = jnp.zeros((_B, _SUB), jnp.int32)

    a_const = a_ref[...]
    col_const = col_ref[...]

    def make_body(masked):
        def body(j, _):
            b = i * _CHUNK + j * _SUB
            x1 = a_const + b.astype(jnp.uint32)
            g = _gumbel_from_counter(x1)
            v = x_ref[:, pl.ds(j * _SUB, _SUB)] + g
            upd = v > accv[...]
            if masked:
                upd = upd & (col_const + b < _V)
            t = i * _NSUB + j
            accv[...] = jnp.where(upd, v, accv[...])
            acct[...] = jnp.where(upd, t, acct[...])
            return 0

        return body

    @pl.when(i < _GRID - 1)
    def _():
        jax.lax.fori_loop(0, _NSUB, make_body(False), 0, unroll=_UNROLL)

    @pl.when(i == _GRID - 1)
    def _():
        jax.lax.fori_loop(0, _NSUB, make_body(True), 0, unroll=_UNROLL)

        av = accv[...]
        colg = acct[...] * _SUB + col_const
        m = jnp.max(av, axis=1, keepdims=True)
        idx = jnp.min(
            jnp.where(av == m, colg, jnp.int32(2**30)),
            axis=1,
            keepdims=True,
        )
        out_ref[...] = idx


@jax.jit
def kernel(logits):
    out = pl.pallas_call(
        _sample_kernel,
        grid=(_GRID,),
        in_specs=[
            pl.BlockSpec((_B, _CHUNK), lambda i: (0, i)),
            pl.BlockSpec((_B, _SUB), lambda i: (0, 0)),
            pl.BlockSpec((_B, _SUB), lambda i: (0, 0)),
        ],
        out_specs=pl.BlockSpec((_B, 1), lambda i: (0, 0)),
        out_shape=jax.ShapeDtypeStruct((_B, 1), jnp.int32),
        scratch_shapes=[
            pltpu.VMEM((_B, _SUB), jnp.float32),
            pltpu.VMEM((_B, _SUB), jnp.int32),
        ],
    )(logits, jnp.asarray(_A_CONST), jnp.asarray(_COL_CONST))
    return out[:, 0].astype(jnp.int64)


# speedup vs baseline: 1.0339x; 1.0339x over previous
import numpy as np
import jax
import jax.numpy as jnp
from jax.experimental import pallas as pl
from jax.experimental.pallas import tpu as pltpu

_K0 = 928981903
_K1 = 3453687069
_K2 = (_K0 ^ _K1 ^ 0x1BD11BDA) & 0xFFFFFFFF

_B = 32
_V = 1_000_000
_CHUNK = 32768
_SUB = 512
_NSUB = _CHUNK // _SUB
_UNROLL = 32
_GRID = (_V + _CHUNK - 1) // _CHUNK

_ROT_A = (13, 15, 26, 6)
_ROT_B = (17, 29, 16, 24)

_KS = (_K0, _K1, _K2)
_SCHED = tuple(
    (np.uint32(_KS[(g + 1) % 3]), np.uint32((_KS[(g + 2) % 3] + g + 1) & 0xFFFFFFFF))
    for g in range(5)
)

_A_NP = (
    np.arange(_B, dtype=np.uint64)[:, None] * _V
    + np.arange(_SUB, dtype=np.uint64)[None, :]
    + _K1
) & 0xFFFFFFFF
_A_CONST = _A_NP.astype(np.uint32)
_COL_CONST = np.broadcast_to(
    np.arange(_SUB, dtype=np.int32)[None, :], (_B, _SUB)
).copy()


def _gumbel_from_counter(x1):
    x0 = x1 + jnp.uint32(_K0)
    first = True
    for g in range(5):
        rots = _ROT_A if g % 2 == 0 else _ROT_B
        for r in rots:
            if first:
                first = False
            else:
                x0 = x0 + x1
            t = (x1 << jnp.uint32(r)) | (x1 >> jnp.uint32(32 - r))
            x1 = x0 ^ t
        c0, c1 = _SCHED[g]
        x0 = x0 + c0
        x1 = x1 + c1
    bits = x0 ^ x1
    fb = (bits >> jnp.uint32(9)) | jnp.uint32(0x3F800000)
    f = jax.lax.bitcast_convert_type(fb, jnp.float32) - jnp.float32(1.0)
    u = f + jnp.float32(1e-20)
    return -jnp.log(-jnp.log(u))


def _sample_kernel(x_ref, a_ref, col_ref, out_ref, accv, acct):
    i = pl.program_id(0)

    @pl.when(i == 0)
    def _():
        accv[...] = jnp.full((_B, _SUB), -jnp.inf, jnp.float32)
        acct[...] = jnp.zeros((_B, _SUB), jnp.int32)

    a_const = a_ref[...]
    col_const = col_ref[...]

    def make_body(masked):
        def body(j, _):
            b = i * _CHUNK + j * _SUB
            x1 = a_const + b.astype(jnp.uint32)
            g = _gumbel_from_counter(x1)
            v = x_ref[:, pl.ds(j * _SUB, _SUB)] + g
            upd = v > accv[...]
            if masked:
                upd = upd & (col_const + b < _V)
            t = i * _NSUB + j
            accv[...] = jnp.where(upd, v, accv[...])
            acct[...] = jnp.where(upd, t, acct[...])
            return 0

        return body

    @pl.when(i < _GRID - 1)
    def _():
        jax.lax.fori_loop(0, _NSUB, make_body(False), 0, unroll=_UNROLL)

    @pl.when(i == _GRID - 1)
    def _():
        jax.lax.fori_loop(0, _NSUB, make_body(True), 0, unroll=_UNROLL)

        av = accv[...]
        colg = acct[...] * _SUB + col_const
        m = jnp.max(av, axis=1, keepdims=True)
        idx = jnp.min(
            jnp.where(av == m, colg, jnp.int32(2**30)),
            axis=1,
            keepdims=True,
        )
        out_ref[...] = idx


@jax.jit
def kernel(logits):
    out = pl.pallas_call(
        _sample_kernel,
        grid=(_GRID,),
        in_specs=[
            pl.BlockSpec((_B, _CHUNK), lambda i: (0, i)),
            pl.BlockSpec((_B, _SUB), lambda i: (0, 0)),
            pl.BlockSpec((_B, _SUB), lambda i: (0, 0)),
        ],
        out_specs=pl.BlockSpec((_B, 1), lambda i: (0, 0)),
        out_shape=jax.ShapeDtypeStruct((_B, 1), jnp.int32),
        scratch_shapes=[
            pltpu.VMEM((_B, _SUB), jnp.float32),
            pltpu.VMEM((_B, _SUB), jnp.int32),
        ],
    )(logits, jnp.asarray(_A_CONST), jnp.asarray(_COL_CONST))
    return out[:, 0].astype(jnp.int64)
